# baseline (speedup 1.0000x reference)
import functools
import jax
import jax.numpy as jnp
from jax import lax
from jax.experimental import pallas as pl
from jax.experimental.pallas import tpu as pltpu
from jax.experimental.pallas import tpu_sc as plsc

N = 10000
E = 320000
D_IN = 128
WIDTH = 256

NC, NS, L = 2, 16, 16
NW = NC * NS
N_PAD = 10240
RPS = N_PAD // NS
C = 128
NCHUNK = 80
EPW = NCHUNK * C
E_PAD = NW * EPW

_mesh = lambda: plsc.VectorSubcoreMesh(core_axis_name="c", subcore_axis_name="s")


def _zero_vmem_2d(ref, rows, cols):
    def body(i, _):
        r = i // (cols // L)
        k = i % (cols // L)
        ref[r, pl.ds(k * L, L)] = jnp.zeros((L,), jnp.float32)
        return 0
    lax.fori_loop(0, rows * (cols // L), body, 0)


@functools.partial(
    pl.kernel,
    out_type=jax.ShapeDtypeStruct((NC, N_PAD), jnp.float32),
    mesh=_mesh(),
    scratch_types=[
        pltpu.VMEM((NCHUNK, 2, C), jnp.int32),
        pltpu.VMEM((C,), jnp.float32),
        pltpu.VMEM((16, D_IN), jnp.float32),
        pltpu.SemaphoreType.DMA,
        pltpu.VMEM_SHARED((N_PAD,), jnp.float32),
    ],
)
def _deg_kernel(eidx_hbm, out_hbm, ei_all, ones_v, slab_v, sem, deg_sh):
    cid = lax.axis_index("c")
    sid = lax.axis_index("s")
    wid = cid * NS + sid
    _zero_vmem_2d(slab_v, 16, D_IN)
    def zc(k, _):
        pltpu.sync_copy(slab_v.at[0], deg_sh.at[pl.ds(sid * RPS + k * D_IN, D_IN)])
        return 0
    lax.fori_loop(0, RPS // D_IN, zc, 0)
    def fill(i, _):
        ones_v[pl.ds(i * L, L)] = jnp.ones((L,), jnp.float32)
        return 0
    lax.fori_loop(0, C // L, fill, 0)
    pltpu.sync_copy(eidx_hbm.at[wid], ei_all)
    plsc.subcore_barrier()

    GB = 8
    def group(g, _):
        descs = [pltpu.async_copy(ones_v, deg_sh.at[ei_all.at[g * GB + j, 1]],
                                  sem, add=True) for j in range(GB)]
        for d in descs:
            d.wait()
        return 0
    lax.fori_loop(0, NCHUNK // GB, group, 0)
    plsc.subcore_barrier()
    pltpu.sync_copy(deg_sh.at[pl.ds(sid * RPS, RPS)],
                    out_hbm.at[cid, pl.ds(sid * RPS, RPS)])


@functools.partial(
    pl.kernel,
    out_type=(
        jax.ShapeDtypeStruct((NC, N_PAD, D_IN), jnp.float32),
        jax.ShapeDtypeStruct((NC, N_PAD), jnp.float32),
    ),
    mesh=_mesh(),
    scratch_types=[
        pltpu.VMEM((4, 2, C), jnp.int32),
        pltpu.VMEM((C, D_IN), jnp.float32),
        pltpu.VMEM((C, D_IN), jnp.float32),
        pltpu.VMEM((C,), jnp.float32),
        pltpu.VMEM((C,), jnp.float32),
        pltpu.VMEM((16, D_IN), jnp.float32),
        pltpu.SemaphoreType.DMA,
        pltpu.SemaphoreType.DMA,
        pltpu.SemaphoreType.DMA,
        pltpu.SemaphoreType.DMA,
        pltpu.SemaphoreType.DMA,
        pltpu.SemaphoreType.DMA,
        pltpu.SemaphoreType.DMA,
        pltpu.SemaphoreType.DMA,
        pltpu.SemaphoreType.DMA,
        pltpu.SemaphoreType.DMA,
        pltpu.SemaphoreType.DMA,
        pltpu.SemaphoreType.DMA,
        pltpu.VMEM_SHARED((N_PAD, D_IN), jnp.float32),
        pltpu.VMEM_SHARED((N_PAD,), jnp.float32),
        pltpu.VMEM_SHARED((N_PAD,), jnp.float32),
    ],
)
def _edge_kernel(eidx_hbm, m_hbm, norm_hbm, agg_out, c_out,
                 ei_ring, rows0, rows1, vals0, vals1, slab_v,
                 si0, si1, si2, si3, sg0, sg1, vg0, vg1,
                 sr0, sr1, vs0, vs1,
                 agg_sh, c_sh, norm_sh):
    cid = lax.axis_index("c")
    sid = lax.axis_index("s")
    wid = cid * NS + sid
    rows = (rows0, rows1)
    vals = (vals0, vals1)
    semi = (si0, si1, si2, si3)
    semg = (sg0, sg1)
    semv = (vg0, vg1)
    semsr = (sr0, sr1)
    semsc = (vs0, vs1)

    _zero_vmem_2d(slab_v, 16, D_IN)
    def zbody(k, _):
        pltpu.sync_copy(slab_v, agg_sh.at[pl.ds(sid * RPS + k * 16, 16)])
        return 0
    lax.fori_loop(0, RPS // 16, zbody, 0)
    def zc(k, _):
        pltpu.sync_copy(slab_v.at[0], c_sh.at[pl.ds(sid * RPS + k * D_IN, D_IN)])
        return 0
    lax.fori_loop(0, RPS // D_IN, zc, 0)
    @pl.when(sid == 0)
    def _():
        pltpu.sync_copy(norm_hbm, norm_sh)
    _zero_vmem_2d(rows1, C, D_IN)
    def zv(i, _):
        vals1[pl.ds(i * L, L)] = jnp.zeros((L,), jnp.float32)
        return 0
    lax.fori_loop(0, C // L, zv, 0)
    plsc.subcore_barrier()

    def i_issue(i, q):
        pltpu.async_copy(eidx_hbm.at[wid, i], ei_ring.at[q], semi[q])

    def i_wait(i, q):
        pltpu.make_async_copy(eidx_hbm.at[wid, i], ei_ring.at[q], semi[q]).wait()

    def g_issue(q, b):
        pltpu.async_copy(m_hbm.at[ei_ring.at[q, 0]], rows[b], semg[b])
        pltpu.async_copy(norm_sh.at[ei_ring.at[q, 1]], vals[b], semv[b])

    def g_wait(q, b):
        pltpu.make_async_copy(m_hbm.at[ei_ring.at[q, 0]], rows[b], semg[b]).wait()
        pltpu.make_async_copy(norm_sh.at[ei_ring.at[q, 1]], vals[b], semv[b]).wait()

    def s_issue(q, b):
        pltpu.async_copy(rows[b], agg_sh.at[ei_ring.at[q, 1]], semsr[b], add=True)
        pltpu.async_copy(vals[b], c_sh.at[ei_ring.at[q, 0]], semsc[b], add=True)

    def s_wait(q, b):
        pltpu.make_async_copy(rows[b], agg_sh.at[ei_ring.at[q, 1]], semsr[b]).wait()
        pltpu.make_async_copy(vals[b], c_sh.at[ei_ring.at[q, 0]], semsc[b]).wait()

    pltpu.sync_copy(eidx_hbm.at[wid, 0], ei_ring.at[0])
    i_issue(1, 1)
    s_issue(0, 1)
    g_issue(0, 0)

    def quad(k, _):
        i0 = 4 * k
        for p in range(4):
            i = i0 + p
            b = p % 2
            q = p
            g_wait(q, b)
            s_issue(q, b)
            s_wait((q - 1) % 4, 1 - b)
            @pl.when(i + 2 < NCHUNK)
            def _():
                i_issue(i + 2, (q + 2) % 4)
            @pl.when(i + 1 < NCHUNK)
            def _():
                i_wait(i + 1, (q + 1) % 4)
                g_issue((q + 1) % 4, 1 - b)
        return 0
    lax.fori_loop(0, NCHUNK // 4, quad, 0)
    s_wait(3, 1)
    plsc.subcore_barrier()

    pltpu.sync_copy(agg_sh.at[pl.ds(sid * RPS, RPS)],
                    agg_out.at[cid, pl.ds(sid * RPS, RPS)])
    pltpu.sync_copy(c_sh.at[pl.ds(sid * RPS, RPS)],
                    c_out.at[cid, pl.ds(sid * RPS, RPS)])


def _tc1_body(d0_ref, d1_ref, x_ref, m_ref, norm_ref):
    deg = d0_ref[...] + d1_ref[...]
    nrm = lax.rsqrt(jnp.maximum(deg, 1.0))
    norm_ref[...] = nrm
    m_ref[...] = x_ref[...] * nrm


BN = 2048
G = N_PAD // BN

def _tc2_body(a0, a1, nrm, c0, c1, W1, b1, W2, b2, out_ref):
    i = pl.program_id(0)
    a = (a0[...] + a1[...]) * nrm[...]
    h = jnp.maximum(
        jnp.dot(a, W1[...], preferred_element_type=jnp.float32) + b1[...], 0.0)
    v = jnp.dot(h, W2[...], preferred_element_type=jnp.float32)
    ridx = lax.broadcasted_iota(jnp.int32, (BN, 1), 0) + i * BN
    cmask = jnp.where(ridx < N, 1.0, 0.0)
    part = jnp.sum(v * nrm[...] * (c0[...] + c1[...]) * cmask)
    prev = jnp.where(i == 0, jnp.zeros((1, 1), jnp.float32), out_ref[...])
    acc = prev + part
    out_ref[...] = jnp.where(i == G - 1, acc / N + b2[...], acc)


def kernel(x, edge_index, W1, b1, W2, b2):
    src = edge_index[0].astype(jnp.int32)
    dst = edge_index[1].astype(jnp.int32)
    fill = jnp.full((E_PAD - E,), N_PAD - 1, jnp.int32)
    srcp = jnp.concatenate([src, fill]).reshape(NW, NCHUNK, 1, C)
    dstp = jnp.concatenate([dst, fill]).reshape(NW, NCHUNK, 1, C)
    eidx = jnp.concatenate([srcp, dstp], axis=2)
    x_pad = jnp.pad(x, ((0, N_PAD - N), (0, 0)))

    deg_parts = _deg_kernel(eidx)
    d0 = deg_parts[0].reshape(N_PAD, 1)
    d1 = deg_parts[1].reshape(N_PAD, 1)

    m, norm_col = pl.pallas_call(
        _tc1_body,
        out_shape=(
            jax.ShapeDtypeStruct((N_PAD, D_IN), jnp.float32),
            jax.ShapeDtypeStruct((N_PAD, 1), jnp.float32),
        ),
    )(d0, d1, x_pad)

    agg_parts, c_parts = _edge_kernel(eidx, m, norm_col.reshape(N_PAD))

    blk = lambda *s: pl.BlockSpec(s, lambda i: (0,) * len(s))
    out2d = pl.pallas_call(
        _tc2_body,
        grid=(G,),
        in_specs=[
            pl.BlockSpec((BN, D_IN), lambda i: (i, 0)),
            pl.BlockSpec((BN, D_IN), lambda i: (i, 0)),
            pl.BlockSpec((BN, 1), lambda i: (i, 0)),
            pl.BlockSpec((BN, 1), lambda i: (i, 0)),
            pl.BlockSpec((BN, 1), lambda i: (i, 0)),
            blk(D_IN, WIDTH), blk(1, WIDTH), blk(WIDTH, 1), blk(1, 1),
        ],
        out_specs=pl.BlockSpec((1, 1), lambda i: (0, 0)),
        out_shape=jax.ShapeDtypeStruct((1, 1), jnp.float32),
    )(agg_parts[0], agg_parts[1], norm_col,
      c_parts[0].reshape(N_PAD, 1), c_parts[1].reshape(N_PAD, 1),
      W1, b1.reshape(1, WIDTH), W2, b2.reshape(1, 1))

    return out2d[0, 0]


# speedup vs baseline: 1.0078x; 1.0078x over previous
import functools
import jax
import jax.numpy as jnp
from jax import lax
from jax.experimental import pallas as pl
from jax.experimental.pallas import tpu as pltpu
from jax.experimental.pallas import tpu_sc as plsc

N = 10000
E = 320000
D_IN = 128
WIDTH = 256

NC, NS, L = 2, 16, 16
NW = NC * NS
N_PAD = 10240
RPS = N_PAD // NS
C = 128
NCHUNK = 80
EPW = NCHUNK * C
E_PAD = NW * EPW

_mesh = lambda: plsc.VectorSubcoreMesh(core_axis_name="c", subcore_axis_name="s")


def _zero_vmem_2d(ref, rows, cols):
    def body(i, _):
        r = i // (cols // L)
        k = i % (cols // L)
        ref[r, pl.ds(k * L, L)] = jnp.zeros((L,), jnp.float32)
        return 0
    lax.fori_loop(0, rows * (cols // L), body, 0)


@functools.partial(
    pl.kernel,
    out_type=jax.ShapeDtypeStruct((NC, N_PAD), jnp.float32),
    mesh=_mesh(),
    scratch_types=[
        pltpu.VMEM((NCHUNK, 2, C), jnp.int32),
        pltpu.VMEM((C,), jnp.float32),
        pltpu.VMEM((16, D_IN), jnp.float32),
        pltpu.SemaphoreType.DMA,
        pltpu.VMEM_SHARED((N_PAD,), jnp.float32),
    ],
)
def _deg_kernel(eidx_hbm, out_hbm, ei_all, ones_v, slab_v, sem, deg_sh):
    cid = lax.axis_index("c")
    sid = lax.axis_index("s")
    wid = cid * NS + sid
    _zero_vmem_2d(slab_v, 16, D_IN)
    def zc(k, _):
        pltpu.sync_copy(slab_v.at[0], deg_sh.at[pl.ds(sid * RPS + k * D_IN, D_IN)])
        return 0
    lax.fori_loop(0, RPS // D_IN, zc, 0)
    def fill(i, _):
        ones_v[pl.ds(i * L, L)] = jnp.ones((L,), jnp.float32)
        return 0
    lax.fori_loop(0, C // L, fill, 0)
    pltpu.sync_copy(eidx_hbm.at[wid], ei_all)
    plsc.subcore_barrier()

    GB = 8
    def group(g, _):
        descs = [pltpu.async_copy(ones_v, deg_sh.at[ei_all.at[g * GB + j, 1]],
                                  sem, add=True) for j in range(GB)]
        for d in descs:
            d.wait()
        return 0
    lax.fori_loop(0, NCHUNK // GB, group, 0)
    plsc.subcore_barrier()
    pltpu.sync_copy(deg_sh.at[pl.ds(sid * RPS, RPS)],
                    out_hbm.at[cid, pl.ds(sid * RPS, RPS)])


@functools.partial(
    pl.kernel,
    out_type=(
        jax.ShapeDtypeStruct((NC, N_PAD, D_IN), jnp.float32),
        jax.ShapeDtypeStruct((NC, N_PAD), jnp.float32),
    ),
    mesh=_mesh(),
    scratch_types=[
        pltpu.VMEM((4, 2, C), jnp.int32),
        pltpu.VMEM((C, D_IN), jnp.float32),
        pltpu.VMEM((C, D_IN), jnp.float32),
        pltpu.VMEM((C,), jnp.float32),
        pltpu.VMEM((C,), jnp.float32),
        pltpu.VMEM((16, D_IN), jnp.float32),
        pltpu.SemaphoreType.DMA,
        pltpu.SemaphoreType.DMA,
        pltpu.SemaphoreType.DMA,
        pltpu.SemaphoreType.DMA,
        pltpu.SemaphoreType.DMA,
        pltpu.SemaphoreType.DMA,
        pltpu.SemaphoreType.DMA,
        pltpu.SemaphoreType.DMA,
        pltpu.VMEM_SHARED((N_PAD, D_IN), jnp.float32),
        pltpu.VMEM_SHARED((N_PAD,), jnp.float32),
        pltpu.VMEM_SHARED((N_PAD,), jnp.float32),
    ],
)
def _edge_kernel(eidx_hbm, m_hbm, norm_hbm, agg_out, c_out,
                 ei_ring, rows0, rows1, vals0, vals1, slab_v,
                 si0, si1, si2, si3, sg0, sg1, vg0, vg1,
                 agg_sh, c_sh, norm_sh):
    cid = lax.axis_index("c")
    sid = lax.axis_index("s")
    wid = cid * NS + sid
    rows = (rows0, rows1)
    vals = (vals0, vals1)
    semi = (si0, si1, si2, si3)
    semg = (sg0, sg1)
    semv = (vg0, vg1)

    _zero_vmem_2d(slab_v, 16, D_IN)
    def zbody(k, _):
        pltpu.sync_copy(slab_v, agg_sh.at[pl.ds(sid * RPS + k * 16, 16)])
        return 0
    lax.fori_loop(0, RPS // 16, zbody, 0)
    def zc(k, _):
        pltpu.sync_copy(slab_v.at[0], c_sh.at[pl.ds(sid * RPS + k * D_IN, D_IN)])
        return 0
    lax.fori_loop(0, RPS // D_IN, zc, 0)
    @pl.when(sid == 0)
    def _():
        pltpu.sync_copy(norm_hbm, norm_sh)
    plsc.subcore_barrier()

    def i_issue(i, q):
        pltpu.async_copy(eidx_hbm.at[wid, i], ei_ring.at[q], semi[q])

    def i_wait(i, q):
        pltpu.make_async_copy(eidx_hbm.at[wid, i], ei_ring.at[q], semi[q]).wait()

    def g_issue(q, b):
        pltpu.async_copy(m_hbm.at[ei_ring.at[q, 0]], rows[b], semg[b])
        pltpu.async_copy(norm_sh.at[ei_ring.at[q, 1]], vals[b], semv[b])

    def g_wait(q, b):
        pltpu.make_async_copy(m_hbm.at[ei_ring.at[q, 0]], rows[b], semg[b]).wait()
        pltpu.make_async_copy(norm_sh.at[ei_ring.at[q, 1]], vals[b], semv[b]).wait()

    def s_sync(q, b):
        pltpu.sync_copy(rows[b], agg_sh.at[ei_ring.at[q, 1]], add=True)
        pltpu.sync_copy(vals[b], c_sh.at[ei_ring.at[q, 0]], add=True)

    pltpu.sync_copy(eidx_hbm.at[wid, 0], ei_ring.at[0])
    i_issue(1, 1)
    g_issue(0, 0)

    def quad(k, _):
        i0 = 4 * k
        for p in range(4):
            i = i0 + p
            b = p % 2
            q = p
            g_wait(q, b)
            @pl.when(i + 2 < NCHUNK)
            def _():
                i_issue(i + 2, (q + 2) % 4)
            @pl.when(i + 1 < NCHUNK)
            def _():
                i_wait(i + 1, (q + 1) % 4)
                g_issue((q + 1) % 4, 1 - b)
            s_sync(q, b)
        return 0
    lax.fori_loop(0, NCHUNK // 4, quad, 0)
    plsc.subcore_barrier()

    pltpu.sync_copy(agg_sh.at[pl.ds(sid * RPS, RPS)],
                    agg_out.at[cid, pl.ds(sid * RPS, RPS)])
    pltpu.sync_copy(c_sh.at[pl.ds(sid * RPS, RPS)],
                    c_out.at[cid, pl.ds(sid * RPS, RPS)])


def _tc1_body(d0_ref, d1_ref, x_ref, m_ref, norm_ref):
    deg = d0_ref[...] + d1_ref[...]
    nrm = lax.rsqrt(jnp.maximum(deg, 1.0))
    norm_ref[...] = nrm
    m_ref[...] = x_ref[...] * nrm


BN = 2048
G = N_PAD // BN

def _tc2_body(a0, a1, nrm, c0, c1, W1, b1, W2, b2, out_ref):
    i = pl.program_id(0)
    a = (a0[...] + a1[...]) * nrm[...]
    h = jnp.maximum(
        jnp.dot(a, W1[...], preferred_element_type=jnp.float32) + b1[...], 0.0)
    v = jnp.dot(h, W2[...], preferred_element_type=jnp.float32)
    ridx = lax.broadcasted_iota(jnp.int32, (BN, 1), 0) + i * BN
    cmask = jnp.where(ridx < N, 1.0, 0.0)
    part = jnp.sum(v * nrm[...] * (c0[...] + c1[...]) * cmask)
    prev = jnp.where(i == 0, jnp.zeros((1, 1), jnp.float32), out_ref[...])
    acc = prev + part
    out_ref[...] = jnp.where(i == G - 1, acc / N + b2[...], acc)


def kernel(x, edge_index, W1, b1, W2, b2):
    src = edge_index[0].astype(jnp.int32)
    dst = edge_index[1].astype(jnp.int32)
    fill = jnp.full((E_PAD - E,), N_PAD - 1, jnp.int32)
    srcp = jnp.concatenate([src, fill]).reshape(NW, NCHUNK, 1, C)
    dstp = jnp.concatenate([dst, fill]).reshape(NW, NCHUNK, 1, C)
    eidx = jnp.concatenate([srcp, dstp], axis=2)
    x_pad = jnp.pad(x, ((0, N_PAD - N), (0, 0)))

    deg_parts = _deg_kernel(eidx)
    d0 = deg_parts[0].reshape(N_PAD, 1)
    d1 = deg_parts[1].reshape(N_PAD, 1)

    m, norm_col = pl.pallas_call(
        _tc1_body,
        out_shape=(
            jax.ShapeDtypeStruct((N_PAD, D_IN), jnp.float32),
            jax.ShapeDtypeStruct((N_PAD, 1), jnp.float32),
        ),
    )(d0, d1, x_pad)

    agg_parts, c_parts = _edge_kernel(eidx, m, norm_col.reshape(N_PAD))

    blk = lambda *s: pl.BlockSpec(s, lambda i: (0,) * len(s))
    out2d = pl.pallas_call(
        _tc2_body,
        grid=(G,),
        in_specs=[
            pl.BlockSpec((BN, D_IN), lambda i: (i, 0)),
            pl.BlockSpec((BN, D_IN), lambda i: (i, 0)),
            pl.BlockSpec((BN, 1), lambda i: (i, 0)),
            pl.BlockSpec((BN, 1), lambda i: (i, 0)),
            pl.BlockSpec((BN, 1), lambda i: (i, 0)),
            blk(D_IN, WIDTH), blk(1, WIDTH), blk(WIDTH, 1), blk(1, 1),
        ],
        out_specs=pl.BlockSpec((1, 1), lambda i: (0, 0)),
        out_shape=jax.ShapeDtypeStruct((1, 1), jnp.float32),
    )(agg_parts[0], agg_parts[1], norm_col,
      c_parts[0].reshape(N_PAD, 1), c_parts[1].reshape(N_PAD, 1),
      W1, b1.reshape(1, WIDTH), W2, b2.reshape(1, 1))

    return out2d[0, 0]


# speedup vs baseline: 2.6078x; 2.5877x over previous
import functools
import jax
import jax.numpy as jnp
from jax import lax
from jax.experimental import pallas as pl
from jax.experimental.pallas import tpu as pltpu
from jax.experimental.pallas import tpu_sc as plsc

N = 10000
E = 320000
D_IN = 128
WIDTH = 256

NC, NS, L = 2, 16, 16
NW = NC * NS
N_PAD = 10240
RPS = N_PAD // NS
C = 128
NCHUNK = 80
EPW = NCHUNK * C
E_PAD = NW * EPW

_mesh = lambda: plsc.VectorSubcoreMesh(core_axis_name="c", subcore_axis_name="s")


def _zero_vmem_2d(ref, rows, cols):
    def body(i, _):
        r = i // (cols // L)
        k = i % (cols // L)
        ref[r, pl.ds(k * L, L)] = jnp.zeros((L,), jnp.float32)
        return 0
    lax.fori_loop(0, rows * (cols // L), body, 0)


@functools.partial(
    pl.kernel,
    out_type=jax.ShapeDtypeStruct((NC, N_PAD), jnp.float32),
    mesh=_mesh(),
    scratch_types=[
        pltpu.VMEM((NCHUNK, 2, C), jnp.int32),
        pltpu.VMEM((C,), jnp.float32),
        pltpu.VMEM((16, D_IN), jnp.float32),
        pltpu.SemaphoreType.DMA,
        pltpu.VMEM_SHARED((N_PAD,), jnp.float32),
    ],
)
def _deg_kernel(eidx_hbm, out_hbm, ei_all, ones_v, slab_v, sem, deg_sh):
    cid = lax.axis_index("c")
    sid = lax.axis_index("s")
    wid = cid * NS + sid
    _zero_vmem_2d(slab_v, 16, D_IN)
    def zc(k, _):
        pltpu.sync_copy(slab_v.at[0], deg_sh.at[pl.ds(sid * RPS + k * D_IN, D_IN)])
        return 0
    lax.fori_loop(0, RPS // D_IN, zc, 0)
    def fill(i, _):
        ones_v[pl.ds(i * L, L)] = jnp.ones((L,), jnp.float32)
        return 0
    lax.fori_loop(0, C // L, fill, 0)
    pltpu.sync_copy(eidx_hbm.at[wid], ei_all)
    plsc.subcore_barrier()

    GB = 8
    def group(g, _):
        descs = [pltpu.async_copy(ones_v, deg_sh.at[ei_all.at[g * GB + j, 1]],
                                  sem, add=True) for j in range(GB)]
        for d in descs:
            d.wait()
        return 0
    lax.fori_loop(0, NCHUNK // GB, group, 0)
    plsc.subcore_barrier()
    pltpu.sync_copy(deg_sh.at[pl.ds(sid * RPS, RPS)],
                    out_hbm.at[cid, pl.ds(sid * RPS, RPS)])


@functools.partial(
    pl.kernel,
    out_type=(
        jax.ShapeDtypeStruct((NC, N_PAD, D_IN), jnp.float32),
        jax.ShapeDtypeStruct((NC, N_PAD), jnp.float32),
    ),
    mesh=_mesh(),
    scratch_types=[
        pltpu.VMEM((4, 2, C), jnp.int32),
        pltpu.VMEM((C, D_IN), jnp.float32),
        pltpu.VMEM((C, D_IN), jnp.float32),
        pltpu.VMEM((C,), jnp.float32),
        pltpu.VMEM((C,), jnp.float32),
        pltpu.VMEM((16, D_IN), jnp.float32),
        pltpu.SemaphoreType.DMA,
        pltpu.SemaphoreType.DMA,
        pltpu.SemaphoreType.DMA,
        pltpu.SemaphoreType.DMA,
        pltpu.SemaphoreType.DMA,
        pltpu.SemaphoreType.DMA,
        pltpu.SemaphoreType.DMA,
        pltpu.SemaphoreType.DMA,
        pltpu.VMEM_SHARED((N_PAD, D_IN), jnp.float32),
        pltpu.VMEM_SHARED((N_PAD,), jnp.float32),
        pltpu.VMEM_SHARED((N_PAD,), jnp.float32),
    ],
)
def _edge_kernel(eidx_hbm, m_hbm, norm_hbm, agg_out, c_out,
                 ei_ring, rows0, rows1, vals0, vals1, slab_v,
                 si0, si1, si2, si3, sg0, sg1, vg0, vg1,
                 agg_sh, c_sh, norm_sh):
    cid = lax.axis_index("c")
    sid = lax.axis_index("s")
    wid = cid * NS + sid
    rows = (rows0, rows1)
    vals = (vals0, vals1)
    semi = (si0, si1, si2, si3)
    semg = (sg0, sg1)
    semv = (vg0, vg1)

    _zero_vmem_2d(slab_v, 16, D_IN)
    def zbody(k, _):
        pltpu.sync_copy(slab_v, agg_sh.at[pl.ds(sid * RPS + k * 16, 16)])
        return 0
    lax.fori_loop(0, RPS // 16, zbody, 0)
    def zc(k, _):
        pltpu.sync_copy(slab_v.at[0], c_sh.at[pl.ds(sid * RPS + k * D_IN, D_IN)])
        return 0
    lax.fori_loop(0, RPS // D_IN, zc, 0)
    @pl.when(sid == 0)
    def _():
        pltpu.sync_copy(norm_hbm, norm_sh)
    plsc.subcore_barrier()

    def i_issue(i, q):
        pltpu.async_copy(eidx_hbm.at[wid, i], ei_ring.at[q], semi[q])

    def i_wait(i, q):
        pltpu.make_async_copy(eidx_hbm.at[wid, i], ei_ring.at[q], semi[q]).wait()

    def g_issue(q, b):
        pltpu.async_copy(m_hbm.at[ei_ring.at[q, 0]], rows[b], semg[b])
        pltpu.async_copy(norm_sh.at[ei_ring.at[q, 1]], vals[b], semv[b])

    def g_wait(q, b):
        pltpu.make_async_copy(m_hbm.at[ei_ring.at[q, 0]], rows[b], semg[b]).wait()
        pltpu.make_async_copy(norm_sh.at[ei_ring.at[q, 1]], vals[b], semv[b]).wait()

    def s_sync(q, b):
        pltpu.sync_copy(rows[b], agg_sh.at[ei_ring.at[q, 1]], add=True)
        pltpu.sync_copy(vals[b], c_sh.at[ei_ring.at[q, 0]], add=True)

    pltpu.sync_copy(eidx_hbm.at[wid, 0], ei_ring.at[0])
    i_issue(1, 1)
    g_issue(0, 0)

    def quad(k, _):
        i0 = 4 * k
        for p in range(4):
            i = i0 + p
            b = p % 2
            q = p
            g_wait(q, b)
            @pl.when(i + 2 < NCHUNK)
            def _():
                i_issue(i + 2, (q + 2) % 4)
            @pl.when(i + 1 < NCHUNK)
            def _():
                i_wait(i + 1, (q + 1) % 4)
                g_issue((q + 1) % 4, 1 - b)
            s_sync(q, b)
        return 0
    lax.fori_loop(0, NCHUNK // 4, quad, 0)
    plsc.subcore_barrier()

    pltpu.sync_copy(agg_sh.at[pl.ds(sid * RPS, RPS)],
                    agg_out.at[cid, pl.ds(sid * RPS, RPS)])
    pltpu.sync_copy(c_sh.at[pl.ds(sid * RPS, RPS)],
                    c_out.at[cid, pl.ds(sid * RPS, RPS)])


def _tc1_body(d0_ref, d1_ref, x_ref, m_ref, norm_ref):
    deg = d0_ref[...] + d1_ref[...]
    nrm = lax.rsqrt(jnp.maximum(deg, 1.0))
    norm_ref[...] = nrm
    m_ref[...] = x_ref[...] * nrm


BN = 2048
G = N_PAD // BN

def _tc2_body(a0, a1, nrm, c0, c1, W1, b1, W2, b2, out_ref):
    i = pl.program_id(0)
    a = (a0[...] + a1[...]) * nrm[...]
    h = jnp.maximum(
        jnp.dot(a, W1[...], preferred_element_type=jnp.float32) + b1[...], 0.0)
    v = jnp.dot(h, W2[...], preferred_element_type=jnp.float32)
    ridx = lax.broadcasted_iota(jnp.int32, (BN, 1), 0) + i * BN
    cmask = jnp.where(ridx < N, 1.0, 0.0)
    part = jnp.sum(v * nrm[...] * (c0[...] + c1[...]) * cmask)
    prev = jnp.where(i == 0, jnp.zeros((1, 1), jnp.float32), out_ref[...])
    acc = prev + part
    out_ref[...] = jnp.where(i == G - 1, acc / N + b2[...], acc)


def kernel(x, edge_index, W1, b1, W2, b2):
    src = edge_index[0].astype(jnp.int32)
    dst = edge_index[1].astype(jnp.int32)
    ppw = EPW - E // NW
    pads = jnp.broadcast_to(jnp.arange(ppw, dtype=jnp.int32) + N, (NW, ppw))
    srcp = jnp.concatenate([src.reshape(NW, E // NW), pads], axis=1)
    dstp = jnp.concatenate([dst.reshape(NW, E // NW), pads], axis=1)
    eidx = jnp.concatenate(
        [srcp.reshape(NW, NCHUNK, 1, C), dstp.reshape(NW, NCHUNK, 1, C)],
        axis=2)
    x_pad = jnp.pad(x, ((0, N_PAD - N), (0, 0)))

    deg_parts = _deg_kernel(eidx)
    d0 = deg_parts[0].reshape(N_PAD, 1)
    d1 = deg_parts[1].reshape(N_PAD, 1)

    m, norm_col = pl.pallas_call(
        _tc1_body,
        out_shape=(
            jax.ShapeDtypeStruct((N_PAD, D_IN), jnp.float32),
            jax.ShapeDtypeStruct((N_PAD, 1), jnp.float32),
        ),
    )(d0, d1, x_pad)

    agg_parts, c_parts = _edge_kernel(eidx, m, norm_col.reshape(N_PAD))

    blk = lambda *s: pl.BlockSpec(s, lambda i: (0,) * len(s))
    out2d = pl.pallas_call(
        _tc2_body,
        grid=(G,),
        in_specs=[
            pl.BlockSpec((BN, D_IN), lambda i: (i, 0)),
            pl.BlockSpec((BN, D_IN), lambda i: (i, 0)),
            pl.BlockSpec((BN, 1), lambda i: (i, 0)),
            pl.BlockSpec((BN, 1), lambda i: (i, 0)),
            pl.BlockSpec((BN, 1), lambda i: (i, 0)),
            blk(D_IN, WIDTH), blk(1, WIDTH), blk(WIDTH, 1), blk(1, 1),
        ],
        out_specs=pl.BlockSpec((1, 1), lambda i: (0, 0)),
        out_shape=jax.ShapeDtypeStruct((1, 1), jnp.float32),
    )(agg_parts[0], agg_parts[1], norm_col,
      c_parts[0].reshape(N_PAD, 1), c_parts[1].reshape(N_PAD, 1),
      W1, b1.reshape(1, WIDTH), W2, b2.reshape(1, 1))

    return out2d[0, 0]


# speedup vs baseline: 2.6626x; 1.0210x over previous
import functools
import jax
import jax.numpy as jnp
from jax import lax
from jax.experimental import pallas as pl
from jax.experimental.pallas import tpu as pltpu
from jax.experimental.pallas import tpu_sc as plsc

N = 10000
E = 320000
D_IN = 128
WIDTH = 256

NC, NS, L = 2, 16, 16
NW = NC * NS
N_PAD = 10240
RPS = N_PAD // NS
C = 128
NCHUNK = 80
EPW = NCHUNK * C
E_PAD = NW * EPW

_mesh = lambda: plsc.VectorSubcoreMesh(core_axis_name="c", subcore_axis_name="s")


def _zero_vmem_2d(ref, rows, cols):
    def body(i, _):
        r = i // (cols // L)
        k = i % (cols // L)
        ref[r, pl.ds(k * L, L)] = jnp.zeros((L,), jnp.float32)
        return 0
    lax.fori_loop(0, rows * (cols // L), body, 0)


@functools.partial(
    pl.kernel,
    out_type=jax.ShapeDtypeStruct((NC, N_PAD), jnp.float32),
    mesh=_mesh(),
    scratch_types=[
        pltpu.VMEM((NCHUNK, 2, C), jnp.int32),
        pltpu.VMEM((C,), jnp.float32),
        pltpu.VMEM((16, D_IN), jnp.float32),
        pltpu.SemaphoreType.DMA,
        pltpu.VMEM_SHARED((N_PAD,), jnp.float32),
    ],
)
def _deg_kernel(eidx_hbm, out_hbm, ei_all, ones_v, slab_v, sem, deg_sh):
    cid = lax.axis_index("c")
    sid = lax.axis_index("s")
    wid = cid * NS + sid
    _zero_vmem_2d(slab_v, 16, D_IN)
    def zc(k, _):
        pltpu.sync_copy(slab_v.at[0], deg_sh.at[pl.ds(sid * RPS + k * D_IN, D_IN)])
        return 0
    lax.fori_loop(0, RPS // D_IN, zc, 0)
    def fill(i, _):
        ones_v[pl.ds(i * L, L)] = jnp.ones((L,), jnp.float32)
        return 0
    lax.fori_loop(0, C // L, fill, 0)
    pltpu.sync_copy(eidx_hbm.at[wid], ei_all)
    plsc.subcore_barrier()

    GB = 8
    def group(g, _):
        descs = [pltpu.async_copy(ones_v, deg_sh.at[ei_all.at[g * GB + j, 1]],
                                  sem, add=True) for j in range(GB)]
        for d in descs:
            d.wait()
        return 0
    lax.fori_loop(0, NCHUNK // GB, group, 0)
    plsc.subcore_barrier()
    pltpu.sync_copy(deg_sh.at[pl.ds(sid * RPS, RPS)],
                    out_hbm.at[cid, pl.ds(sid * RPS, RPS)])


@functools.partial(
    pl.kernel,
    out_type=(
        jax.ShapeDtypeStruct((NC, N_PAD, D_IN), jnp.float32),
        jax.ShapeDtypeStruct((NC, N_PAD), jnp.float32),
    ),
    mesh=_mesh(),
    scratch_types=[
        pltpu.VMEM((4, 2, C), jnp.int32),
        pltpu.VMEM((C, D_IN), jnp.float32),
        pltpu.VMEM((C, D_IN), jnp.float32),
        pltpu.VMEM((C,), jnp.float32),
        pltpu.VMEM((C,), jnp.float32),
        pltpu.VMEM((16, D_IN), jnp.float32),
        pltpu.SemaphoreType.DMA,
        pltpu.SemaphoreType.DMA,
        pltpu.SemaphoreType.DMA,
        pltpu.SemaphoreType.DMA,
        pltpu.SemaphoreType.DMA,
        pltpu.SemaphoreType.DMA,
        pltpu.SemaphoreType.DMA,
        pltpu.SemaphoreType.DMA,
        pltpu.SemaphoreType.DMA,
        pltpu.SemaphoreType.DMA,
        pltpu.SemaphoreType.DMA,
        pltpu.SemaphoreType.DMA,
        pltpu.VMEM_SHARED((N_PAD, D_IN), jnp.float32),
        pltpu.VMEM_SHARED((N_PAD,), jnp.float32),
        pltpu.VMEM_SHARED((N_PAD,), jnp.float32),
    ],
)
def _edge_kernel(eidx_hbm, m_hbm, norm_hbm, agg_out, c_out,
                 ei_ring, rows0, rows1, vals0, vals1, slab_v,
                 si0, si1, si2, si3, sg0, sg1, vg0, vg1,
                 sr0, sr1, vs0, vs1,
                 agg_sh, c_sh, norm_sh):
    cid = lax.axis_index("c")
    sid = lax.axis_index("s")
    wid = cid * NS + sid
    rows = (rows0, rows1)
    vals = (vals0, vals1)
    semi = (si0, si1, si2, si3)
    semg = (sg0, sg1)
    semv = (vg0, vg1)
    semsr = (sr0, sr1)
    semsc = (vs0, vs1)

    _zero_vmem_2d(slab_v, 16, D_IN)
    def zbody(k, _):
        pltpu.sync_copy(slab_v, agg_sh.at[pl.ds(sid * RPS + k * 16, 16)])
        return 0
    lax.fori_loop(0, RPS // 16, zbody, 0)
    def zc(k, _):
        pltpu.sync_copy(slab_v.at[0], c_sh.at[pl.ds(sid * RPS + k * D_IN, D_IN)])
        return 0
    lax.fori_loop(0, RPS // D_IN, zc, 0)
    @pl.when(sid == 0)
    def _():
        pltpu.sync_copy(norm_hbm, norm_sh)
    _zero_vmem_2d(rows1, C, D_IN)
    def zv(i, _):
        vals1[pl.ds(i * L, L)] = jnp.zeros((L,), jnp.float32)
        return 0
    lax.fori_loop(0, C // L, zv, 0)
    plsc.subcore_barrier()

    def i_issue(i, q):
        pltpu.async_copy(eidx_hbm.at[wid, i], ei_ring.at[q], semi[q])

    def i_wait(i, q):
        pltpu.make_async_copy(eidx_hbm.at[wid, i], ei_ring.at[q], semi[q]).wait()

    def g_issue(q, b):
        pltpu.async_copy(m_hbm.at[ei_ring.at[q, 0]], rows[b], semg[b])
        pltpu.async_copy(norm_sh.at[ei_ring.at[q, 1]], vals[b], semv[b])

    def g_wait(q, b):
        pltpu.make_async_copy(m_hbm.at[ei_ring.at[q, 0]], rows[b], semg[b]).wait()
        pltpu.make_async_copy(norm_sh.at[ei_ring.at[q, 1]], vals[b], semv[b]).wait()

    def s_issue(q, b):
        pltpu.async_copy(rows[b], agg_sh.at[ei_ring.at[q, 1]], semsr[b], add=True)
        pltpu.async_copy(vals[b], c_sh.at[ei_ring.at[q, 0]], semsc[b], add=True)

    def s_wait(q, b):
        pltpu.make_async_copy(rows[b], agg_sh.at[ei_ring.at[q, 1]], semsr[b]).wait()
        pltpu.make_async_copy(vals[b], c_sh.at[ei_ring.at[q, 0]], semsc[b]).wait()

    pltpu.sync_copy(eidx_hbm.at[wid, 0], ei_ring.at[0])
    i_issue(1, 1)
    s_issue(0, 1)
    g_issue(0, 0)

    def quad(k, _):
        i0 = 4 * k
        for p in range(4):
            i = i0 + p
            b = p % 2
            q = p
            g_wait(q, b)
            s_issue(q, b)
            s_wait((q - 1) % 4, 1 - b)
            @pl.when(i + 2 < NCHUNK)
            def _():
                i_issue(i + 2, (q + 2) % 4)
            @pl.when(i + 1 < NCHUNK)
            def _():
                i_wait(i + 1, (q + 1) % 4)
                g_issue((q + 1) % 4, 1 - b)
        return 0
    lax.fori_loop(0, NCHUNK // 4, quad, 0)
    s_wait(3, 1)
    plsc.subcore_barrier()

    pltpu.sync_copy(agg_sh.at[pl.ds(sid * RPS, RPS)],
                    agg_out.at[cid, pl.ds(sid * RPS, RPS)])
    pltpu.sync_copy(c_sh.at[pl.ds(sid * RPS, RPS)],
                    c_out.at[cid, pl.ds(sid * RPS, RPS)])


def _tc1_body(dp_ref, x_ref, m_ref, norm_ref):
    dp = dp_ref[...]
    nrm = lax.rsqrt(jnp.maximum(dp[0] + dp[1], 1.0))
    norm_ref[...] = nrm
    m_ref[pl.ds(0, N), :] = x_ref[...] * nrm[:N]
    m_ref[pl.ds(N, N_PAD - N), :] = jnp.zeros((N_PAD - N, D_IN), jnp.float32)


BN = 2048
G = N_PAD // BN

def _tc2_body(ap, nrm, cp, W1, b1, W2, b2, out_ref):
    i = pl.program_id(0)
    a3 = ap[...]
    a = (a3[0] + a3[1]) * nrm[...]
    h = jnp.maximum(
        jnp.dot(a, W1[...], preferred_element_type=jnp.float32) + b1[...], 0.0)
    v = jnp.dot(h, W2[...], preferred_element_type=jnp.float32)
    ridx = lax.broadcasted_iota(jnp.int32, (BN, 1), 0) + i * BN
    cmask = jnp.where(ridx < N, 1.0, 0.0)
    c3 = cp[...]
    part = jnp.sum(v * nrm[...] * (c3[0] + c3[1]) * cmask)
    prev = jnp.where(i == 0, jnp.zeros((1, 1), jnp.float32), out_ref[...])
    acc = prev + part
    out_ref[...] = jnp.where(i == G - 1, acc / N + b2[...], acc)


def kernel(x, edge_index, W1, b1, W2, b2):
    src = edge_index[0].astype(jnp.int32)
    dst = edge_index[1].astype(jnp.int32)
    ppw = EPW - E // NW
    pads = jnp.broadcast_to(jnp.arange(ppw, dtype=jnp.int32) + N, (NW, ppw))
    srcp = jnp.concatenate([src.reshape(NW, E // NW), pads], axis=1)
    dstp = jnp.concatenate([dst.reshape(NW, E // NW), pads], axis=1)
    eidx = jnp.concatenate(
        [srcp.reshape(NW, NCHUNK, 1, C), dstp.reshape(NW, NCHUNK, 1, C)],
        axis=2)

    deg_parts = _deg_kernel(eidx)

    m, norm_col = pl.pallas_call(
        _tc1_body,
        out_shape=(
            jax.ShapeDtypeStruct((N_PAD, D_IN), jnp.float32),
            jax.ShapeDtypeStruct((N_PAD, 1), jnp.float32),
        ),
    )(deg_parts.reshape(NC, N_PAD, 1), x)

    agg_parts, c_parts = _edge_kernel(eidx, m, norm_col.reshape(N_PAD))

    blk = lambda *s: pl.BlockSpec(s, lambda i: (0,) * len(s))
    out2d = pl.pallas_call(
        _tc2_body,
        grid=(G,),
        in_specs=[
            pl.BlockSpec((2, BN, D_IN), lambda i: (0, i, 0)),
            pl.BlockSpec((BN, 1), lambda i: (i, 0)),
            pl.BlockSpec((2, BN, 1), lambda i: (0, i, 0)),
            blk(D_IN, WIDTH), blk(1, WIDTH), blk(WIDTH, 1), blk(1, 1),
        ],
        out_specs=pl.BlockSpec((1, 1), lambda i: (0, 0)),
        out_shape=jax.ShapeDtypeStruct((1, 1), jnp.float32),
    )(agg_parts, norm_col, c_parts.reshape(NC, N_PAD, 1),
      W1, b1.reshape(1, WIDTH), W2, b2.reshape(1, 1))

    return out2d[0, 0]


# speedup vs baseline: 2.6998x; 1.0140x over previous
import functools
import jax
import jax.numpy as jnp
from jax import lax
from jax.experimental import pallas as pl
from jax.experimental.pallas import tpu as pltpu
from jax.experimental.pallas import tpu_sc as plsc

N = 10000
E = 320000
D_IN = 128
WIDTH = 256

NC, NS, L = 2, 16, 16
NW = NC * NS
N_PAD = 10240
RPS = N_PAD // NS
C = 128
NCHUNK = 80
EPW = NCHUNK * C
E_PAD = NW * EPW

_mesh = lambda: plsc.VectorSubcoreMesh(core_axis_name="c", subcore_axis_name="s")


def _zero_vmem_2d(ref, rows, cols):
    def body(i, _):
        r = i // (cols // L)
        k = i % (cols // L)
        ref[r, pl.ds(k * L, L)] = jnp.zeros((L,), jnp.float32)
        return 0
    lax.fori_loop(0, rows * (cols // L), body, 0)


@functools.partial(
    pl.kernel,
    out_type=jax.ShapeDtypeStruct((NC, N_PAD), jnp.float32),
    mesh=_mesh(),
    scratch_types=[
        pltpu.VMEM((NCHUNK, 2, C), jnp.int32),
        pltpu.VMEM((C,), jnp.float32),
        pltpu.VMEM((16, D_IN), jnp.float32),
        pltpu.SemaphoreType.DMA,
        pltpu.VMEM_SHARED((N_PAD,), jnp.float32),
    ],
)
def _deg_kernel(eidx_hbm, out_hbm, ei_all, ones_v, slab_v, sem, deg_sh):
    cid = lax.axis_index("c")
    sid = lax.axis_index("s")
    wid = cid * NS + sid
    _zero_vmem_2d(slab_v, 16, D_IN)
    def zc(k, _):
        pltpu.sync_copy(slab_v.at[0], deg_sh.at[pl.ds(sid * RPS + k * D_IN, D_IN)])
        return 0
    lax.fori_loop(0, RPS // D_IN, zc, 0)
    def fill(i, _):
        ones_v[pl.ds(i * L, L)] = jnp.ones((L,), jnp.float32)
        return 0
    lax.fori_loop(0, C // L, fill, 0)
    pltpu.sync_copy(eidx_hbm.at[wid], ei_all)
    plsc.subcore_barrier()

    GB = 16
    def group(g, _):
        descs = [pltpu.async_copy(ones_v, deg_sh.at[ei_all.at[g * GB + j, 1]],
                                  sem, add=True) for j in range(GB)]
        for d in descs:
            d.wait()
        return 0
    lax.fori_loop(0, NCHUNK // GB, group, 0)
    plsc.subcore_barrier()
    pltpu.sync_copy(deg_sh.at[pl.ds(sid * RPS, RPS)],
                    out_hbm.at[cid, pl.ds(sid * RPS, RPS)])


@functools.partial(
    pl.kernel,
    out_type=(
        jax.ShapeDtypeStruct((NC, N_PAD, D_IN), jnp.float32),
        jax.ShapeDtypeStruct((NC, N_PAD), jnp.float32),
    ),
    mesh=_mesh(),
    scratch_types=[
        pltpu.VMEM((4, 2, C), jnp.int32),
        pltpu.VMEM((C, D_IN), jnp.float32),
        pltpu.VMEM((C, D_IN), jnp.float32),
        pltpu.VMEM((C,), jnp.float32),
        pltpu.VMEM((C,), jnp.float32),
        pltpu.VMEM((16, D_IN), jnp.float32),
        pltpu.SemaphoreType.DMA,
        pltpu.SemaphoreType.DMA,
        pltpu.SemaphoreType.DMA,
        pltpu.SemaphoreType.DMA,
        pltpu.SemaphoreType.DMA,
        pltpu.SemaphoreType.DMA,
        pltpu.SemaphoreType.DMA,
        pltpu.SemaphoreType.DMA,
        pltpu.SemaphoreType.DMA,
        pltpu.SemaphoreType.DMA,
        pltpu.SemaphoreType.DMA,
        pltpu.SemaphoreType.DMA,
        pltpu.VMEM_SHARED((N_PAD, D_IN), jnp.float32),
        pltpu.VMEM_SHARED((N_PAD,), jnp.float32),
        pltpu.VMEM_SHARED((N_PAD,), jnp.float32),
    ],
)
def _edge_kernel(eidx_hbm, m_hbm, norm_hbm, agg_out, c_out,
                 ei_ring, rows0, rows1, vals0, vals1, slab_v,
                 si0, si1, si2, si3, sg0, sg1, vg0, vg1,
                 sr0, sr1, vs0, vs1,
                 agg_sh, c_sh, norm_sh):
    cid = lax.axis_index("c")
    sid = lax.axis_index("s")
    wid = cid * NS + sid
    rows = (rows0, rows1)
    vals = (vals0, vals1)
    semi = (si0, si1, si2, si3)
    semg = (sg0, sg1)
    semv = (vg0, vg1)
    semsr = (sr0, sr1)
    semsc = (vs0, vs1)

    _zero_vmem_2d(slab_v, 16, D_IN)
    def zbody(k, _):
        descs = [pltpu.async_copy(
            slab_v, agg_sh.at[pl.ds(sid * RPS + (8 * k + j) * 16, 16)], sg0)
            for j in range(8)]
        for d in descs:
            d.wait()
        return 0
    lax.fori_loop(0, RPS // 128, zbody, 0)
    def zc(k, _):
        pltpu.sync_copy(slab_v.at[0], c_sh.at[pl.ds(sid * RPS + k * D_IN, D_IN)])
        return 0
    lax.fori_loop(0, RPS // D_IN, zc, 0)
    @pl.when(sid == 0)
    def _():
        pltpu.sync_copy(norm_hbm, norm_sh)
    _zero_vmem_2d(rows1, C, D_IN)
    def zv(i, _):
        vals1[pl.ds(i * L, L)] = jnp.zeros((L,), jnp.float32)
        return 0
    lax.fori_loop(0, C // L, zv, 0)
    plsc.subcore_barrier()

    def i_issue(i, q):
        pltpu.async_copy(eidx_hbm.at[wid, i], ei_ring.at[q], semi[q])

    def i_wait(i, q):
        pltpu.make_async_copy(eidx_hbm.at[wid, i], ei_ring.at[q], semi[q]).wait()

    def g_issue(q, b):
        pltpu.async_copy(m_hbm.at[ei_ring.at[q, 0]], rows[b], semg[b])
        pltpu.async_copy(norm_sh.at[ei_ring.at[q, 1]], vals[b], semv[b])

    def g_wait(q, b):
        pltpu.make_async_copy(m_hbm.at[ei_ring.at[q, 0]], rows[b], semg[b]).wait()
        pltpu.make_async_copy(norm_sh.at[ei_ring.at[q, 1]], vals[b], semv[b]).wait()

    def s_issue(q, b):
        pltpu.async_copy(rows[b], agg_sh.at[ei_ring.at[q, 1]], semsr[b], add=True)
        pltpu.async_copy(vals[b], c_sh.at[ei_ring.at[q, 0]], semsc[b], add=True)

    def s_wait(q, b):
        pltpu.make_async_copy(rows[b], agg_sh.at[ei_ring.at[q, 1]], semsr[b]).wait()
        pltpu.make_async_copy(vals[b], c_sh.at[ei_ring.at[q, 0]], semsc[b]).wait()

    pltpu.sync_copy(eidx_hbm.at[wid, 0], ei_ring.at[0])
    i_issue(1, 1)
    s_issue(0, 1)
    g_issue(0, 0)

    def quad(k, _):
        i0 = 4 * k
        for p in range(4):
            i = i0 + p
            b = p % 2
            q = p
            g_wait(q, b)
            s_issue(q, b)
            s_wait((q - 1) % 4, 1 - b)
            @pl.when(i + 2 < NCHUNK)
            def _():
                i_issue(i + 2, (q + 2) % 4)
            @pl.when(i + 1 < NCHUNK)
            def _():
                i_wait(i + 1, (q + 1) % 4)
                g_issue((q + 1) % 4, 1 - b)
        return 0
    lax.fori_loop(0, NCHUNK // 4, quad, 0)
    s_wait(3, 1)
    plsc.subcore_barrier()

    pltpu.sync_copy(agg_sh.at[pl.ds(sid * RPS, RPS)],
                    agg_out.at[cid, pl.ds(sid * RPS, RPS)])
    pltpu.sync_copy(c_sh.at[pl.ds(sid * RPS, RPS)],
                    c_out.at[cid, pl.ds(sid * RPS, RPS)])


def _tc1_body(dp_ref, x_ref, m_ref, norm_ref):
    dp = dp_ref[...]
    nrm = lax.rsqrt(jnp.maximum(dp[0] + dp[1], 1.0))
    norm_ref[...] = nrm
    m_ref[pl.ds(0, N), :] = x_ref[...] * nrm[:N]
    m_ref[pl.ds(N, N_PAD - N), :] = jnp.zeros((N_PAD - N, D_IN), jnp.float32)


BN = 2048
G = N_PAD // BN

def _tc2_body(ap, nrm, cp, W1, b1, W2, b2, out_ref):
    i = pl.program_id(0)
    a3 = ap[...]
    a = (a3[0] + a3[1]) * nrm[...]
    h = jnp.maximum(
        jnp.dot(a, W1[...], preferred_element_type=jnp.float32) + b1[...], 0.0)
    v = jnp.dot(h, W2[...], preferred_element_type=jnp.float32)
    ridx = lax.broadcasted_iota(jnp.int32, (BN, 1), 0) + i * BN
    cmask = jnp.where(ridx < N, 1.0, 0.0)
    c3 = cp[...]
    part = jnp.sum(v * nrm[...] * (c3[0] + c3[1]) * cmask)
    prev = jnp.where(i == 0, jnp.zeros((1, 1), jnp.float32), out_ref[...])
    acc = prev + part
    out_ref[...] = jnp.where(i == G - 1, acc / N + b2[...], acc)


def kernel(x, edge_index, W1, b1, W2, b2):
    src = edge_index[0].astype(jnp.int32)
    dst = edge_index[1].astype(jnp.int32)
    ppw = EPW - E // NW
    pads = jnp.broadcast_to(jnp.arange(ppw, dtype=jnp.int32) + N, (NW, ppw))
    srcp = jnp.concatenate([src.reshape(NW, E // NW), pads], axis=1)
    dstp = jnp.concatenate([dst.reshape(NW, E // NW), pads], axis=1)
    eidx = jnp.concatenate(
        [srcp.reshape(NW, NCHUNK, 1, C), dstp.reshape(NW, NCHUNK, 1, C)],
        axis=2)

    deg_parts = _deg_kernel(eidx)

    m, norm_col = pl.pallas_call(
        _tc1_body,
        out_shape=(
            jax.ShapeDtypeStruct((N_PAD, D_IN), jnp.float32),
            jax.ShapeDtypeStruct((N_PAD, 1), jnp.float32),
        ),
    )(deg_parts.reshape(NC, N_PAD, 1), x)

    agg_parts, c_parts = _edge_kernel(eidx, m, norm_col.reshape(N_PAD))

    blk = lambda *s: pl.BlockSpec(s, lambda i: (0,) * len(s))
    out2d = pl.pallas_call(
        _tc2_body,
        grid=(G,),
        in_specs=[
            pl.BlockSpec((2, BN, D_IN), lambda i: (0, i, 0)),
            pl.BlockSpec((BN, 1), lambda i: (i, 0)),
            pl.BlockSpec((2, BN, 1), lambda i: (0, i, 0)),
            blk(D_IN, WIDTH), blk(1, WIDTH), blk(WIDTH, 1), blk(1, 1),
        ],
        out_specs=pl.BlockSpec((1, 1), lambda i: (0, 0)),
        out_shape=jax.ShapeDtypeStruct((1, 1), jnp.float32),
    )(agg_parts, norm_col, c_parts.reshape(NC, N_PAD, 1),
      W1, b1.reshape(1, WIDTH), W2, b2.reshape(1, 1))

    return out2d[0, 0]
